# baseline (speedup 1.0000x reference)
import functools

import jax
import jax.numpy as jnp
from jax import lax
from jax.experimental import pallas as pl
from jax.experimental.pallas import tpu as pltpu
from jax.experimental.pallas import tpu_sc as plsc


def _make_sc_broadcast(B: int, P: int, D: int, dtype):
    info = plsc.get_sparse_core_info()
    NC, NS = info.num_cores, info.num_subcores
    NW = NC * NS
    assert P % NW == 0, (P, NW)
    rows_per_w = P // NW
    mesh = plsc.VectorSubcoreMesh(core_axis_name="c", subcore_axis_name="s")

    @functools.partial(
        pl.kernel,
        mesh=mesh,
        out_type=jax.ShapeDtypeStruct((B, P, D), dtype),
        scratch_types=[
            pltpu.VMEM((rows_per_w, D), dtype),
            pltpu.SemaphoreType.DMA,
        ],
    )
    def broadcast_kernel(table_hbm, out_hbm, chunk_v, sem):
        wid = lax.axis_index("s") * NC + lax.axis_index("c")
        base = wid * rows_per_w
        pltpu.sync_copy(table_hbm.at[pl.ds(base, rows_per_w), :], chunk_v)
        copies = [
            pltpu.async_copy(
                chunk_v, out_hbm.at[b, pl.ds(base, rows_per_w), :], sem
            )
            for b in range(B)
        ]
        for cp in copies:
            cp.wait()

    return broadcast_kernel


def kernel(x, pos_embed):
    B, C, H, W = x.shape
    P = H * W
    D = pos_embed.shape[1]
    fn = _make_sc_broadcast(B, P, D, pos_embed.dtype)
    return fn(pos_embed)


# speedup vs baseline: 2.0055x; 2.0055x over previous
import jax
import jax.numpy as jnp
from jax.experimental import pallas as pl
from jax.experimental.pallas import tpu as pltpu


def _tc_broadcast(B: int, P: int, D: int, dtype, table):
    def body(emb_ref, out_ref):
        out_ref[0] = emb_ref[...]

    return pl.pallas_call(
        body,
        grid=(B,),
        in_specs=[pl.BlockSpec((P, D), lambda b: (0, 0))],
        out_specs=pl.BlockSpec((1, P, D), lambda b: (b, 0, 0)),
        out_shape=jax.ShapeDtypeStruct((B, P, D), dtype),
        compiler_params=pltpu.CompilerParams(
            dimension_semantics=("arbitrary",),
        ),
    )(table)


def kernel(x, pos_embed):
    B, C, H, W = x.shape
    P = H * W
    D = pos_embed.shape[1]
    return _tc_broadcast(B, P, D, pos_embed.dtype, pos_embed)
